# baseline (speedup 1.0000x reference)
import functools

import jax
import jax.numpy as jnp
from jax.experimental import pallas as pl
from jax.experimental.pallas import tpu as pltpu

PB = 2048


def _body(ptsT_ref, gb_ref, gbT_ref,
          cls_ref, plx_ref, ply_ref, plz_ref, bidx_ref,
          prep_ref, wg_ref):
    R = gb_ref.shape[0]
    M = 128

    @pl.when(pl.program_id(0) == 0)
    def _prep():
        gb = gb_ref[...]
        ang = -gb[:, 6:7]
        c = jnp.cos(ang)
        s = jnp.sin(ang)
        valid = (gb[:, 3:4] + gb[:, 4:5] + gb[:, 5:6]) > 0.0
        hx = jnp.where(valid, gb[:, 3:4] * 0.5, -1.0)
        hy = gb[:, 4:5] * 0.5
        hz = gb[:, 5:6] * 0.5
        kf = jnp.floor_divide(
            jax.lax.broadcasted_iota(jnp.int32, (R, 1), 0), M
        ).astype(jnp.float32)
        prep_ref[...] = jnp.concatenate(
            [gb[:, 0:1], gb[:, 1:2], gb[:, 2:3], c, s, hx, hy, hz,
             kf, jnp.zeros((R, 7), jnp.float32)], axis=1)
        gbT = gbT_ref[...]
        angT = -gbT[6:7, :]
        wg_ref[0:3, :] = gbT[0:3, :]
        wg_ref[3:4, :] = jnp.cos(angT)
        wg_ref[4:5, :] = jnp.sin(angT)
        wg_ref[5:8, :] = gbT[3:6, :]
        wg_ref[8:9, :] = gbT[7:8, :]
        wg_ref[9:16, :] = jnp.zeros((7, R), jnp.float32)

    pc = prep_ref[...]
    cx = pc[:, 0:1]
    cy = pc[:, 1:2]
    cz = pc[:, 2:3]
    cc = pc[:, 3:4]
    ss = pc[:, 4:5]
    hx = pc[:, 5:6]
    hy = pc[:, 6:7]
    hz = pc[:, 7:8]
    kf = pc[:, 8:9]

    blk = ptsT_ref[...]
    bs = blk[0:1, :]
    xr = blk[1:2, :]
    yr = blk[2:3, :]
    zr = blk[3:4, :]
    pb = blk.shape[1]
    nk = R // M

    iota = jax.lax.broadcasted_iota(jnp.int32, (M, pb), 0).astype(jnp.float32)
    rf = jnp.float32(R)
    fis = []
    for k in range(nk):
        sl = slice(k * M, (k + 1) * M)
        dx = xr - cx[sl]
        dy = yr - cy[sl]
        dz = zr - cz[sl]
        lx = dx * cc[sl] - dy * ss[sl]
        ly = dx * ss[sl] + dy * cc[sl]
        inb = ((jnp.abs(lx) <= hx[sl]) & (jnp.abs(ly) <= hy[sl])
               & (jnp.abs(dz) <= hz[sl]))
        cand = jnp.where(inb, iota, rf)
        mn = jnp.min(cand, axis=0, keepdims=True)
        fis.append(jnp.where(mn < M, mn + (k * M), rf))
    fif = fis[nk - 1]
    for k in range(nk - 2, -1, -1):
        fif = jnp.where(bs == jnp.float32(k), fis[k], fif)
    fg = fif < rf
    fi = fif.astype(jnp.int32)

    G = jnp.zeros((16, pb), jnp.float32)
    for k in range(nk):
        ohf = (iota == (fif - jnp.float32(k * M))).astype(jnp.float32)
        G = G + jax.lax.dot_general(
            wg_ref[:, k * M:(k + 1) * M], ohf, (((1,), (0,)), ((), ())),
            precision=jax.lax.Precision.HIGHEST,
            preferred_element_type=jnp.float32)

    px = xr - G[0:1, :]
    py = yr - G[1:2, :]
    pz = zr - G[2:3, :]
    gc = G[3:4, :]
    gs = G[4:5, :]
    rx = px * gc - py * gs
    ry = px * gs + py * gc
    plx = jnp.where(fg, rx / G[5:6, :] + 0.5, 0.0)
    ply = jnp.where(fg, ry / G[6:7, :] + 0.5, 0.0)
    plz = jnp.where(fg, pz / G[7:8, :] + 0.5, 0.0)

    cls = jnp.where(fg, G[8:9, :].astype(jnp.int32) + 1, 0)
    bidx = jnp.where(fg, fi, -1)

    cls_ref[...] = cls.reshape(1, 1, cls.shape[-1])
    plx_ref[...] = plx.reshape(1, 1, plx.shape[-1])
    ply_ref[...] = ply.reshape(1, 1, ply.shape[-1])
    plz_ref[...] = plz.reshape(1, 1, plz.shape[-1])
    bidx_ref[...] = bidx.reshape(1, 1, bidx.shape[-1])


@functools.partial(jax.jit, static_argnames=())
def kernel(points, gt_boxes):
    n = points.shape[0]
    b, m, _ = gt_boxes.shape
    r = b * m
    pb = PB if n % PB == 0 else n
    g = n // pb

    ptsT = jnp.transpose(points)
    gb = gt_boxes.reshape(r, 8)
    gbT = jnp.transpose(gb)

    grid = (g,)
    out_shapes = [
        jax.ShapeDtypeStruct((g, 1, pb), jnp.int32),
        jax.ShapeDtypeStruct((g, 1, pb), jnp.float32),
        jax.ShapeDtypeStruct((g, 1, pb), jnp.float32),
        jax.ShapeDtypeStruct((g, 1, pb), jnp.float32),
        jax.ShapeDtypeStruct((g, 1, pb), jnp.int32),
    ]
    out_specs = [pl.BlockSpec((1, 1, pb), lambda i: (i, 0, 0))
                 for _ in range(5)]
    in_specs = [
        pl.BlockSpec((4, pb), lambda i: (0, i)),
        pl.BlockSpec((r, 8), lambda i: (0, 0)),
        pl.BlockSpec((8, r), lambda i: (0, 0)),
    ]
    scratch = [
        pltpu.VMEM((r, 16), jnp.float32),
        pltpu.VMEM((16, r), jnp.float32),
    ]
    cls, plx, ply, plz, bidx = pl.pallas_call(
        _body,
        grid=grid,
        in_specs=in_specs,
        out_specs=out_specs,
        out_shape=out_shapes,
        scratch_shapes=scratch,
    )(ptsT, gb, gbT)

    part = jnp.concatenate(
        [plx.reshape(n, 1), ply.reshape(n, 1), plz.reshape(n, 1)], axis=1)
    return cls.reshape(n), part, bidx.reshape(n)


# speedup vs baseline: 1.0581x; 1.0581x over previous
import functools

import jax
import jax.numpy as jnp
from jax.experimental import pallas as pl
from jax.experimental.pallas import tpu as pltpu

PB = 1024
TB = 256


def _body(ptsT_ref, gb_ref, gbT_ref,
          cls_ref, plx_ref, ply_ref, plz_ref, bidx_ref,
          prep_ref, wg_ref):
    R = gb_ref.shape[0]
    M = 128

    @pl.when(pl.program_id(0) == 0)
    def _prep():
        gb = gb_ref[...]
        ang = -gb[:, 6:7]
        c = jnp.cos(ang)
        s = jnp.sin(ang)
        valid = (gb[:, 3:4] + gb[:, 4:5] + gb[:, 5:6]) > 0.0
        hx = jnp.where(valid, gb[:, 3:4] * 0.5, -1.0)
        hy = gb[:, 4:5] * 0.5
        hz = gb[:, 5:6] * 0.5
        kf = jnp.floor_divide(
            jax.lax.broadcasted_iota(jnp.int32, (R, 1), 0), M
        ).astype(jnp.float32)
        prep_ref[...] = jnp.concatenate(
            [gb[:, 0:1], gb[:, 1:2], gb[:, 2:3], c, s, hx, hy, hz,
             kf, jnp.zeros((R, 7), jnp.float32)], axis=1)
        gbT = gbT_ref[...]
        angT = -gbT[6:7, :]
        wg_ref[0:3, :] = gbT[0:3, :]
        wg_ref[3:4, :] = jnp.cos(angT)
        wg_ref[4:5, :] = jnp.sin(angT)
        wg_ref[5:8, :] = gbT[3:6, :]
        wg_ref[8:9, :] = gbT[7:8, :]
        wg_ref[9:16, :] = jnp.zeros((7, R), jnp.float32)

    pc = prep_ref[...]
    cx = pc[:, 0:1]
    cy = pc[:, 1:2]
    cz = pc[:, 2:3]
    cc = pc[:, 3:4]
    ss = pc[:, 4:5]
    hx = pc[:, 5:6]
    hy = pc[:, 6:7]
    hz = pc[:, 7:8]
    kf = pc[:, 8:9]

    blk = ptsT_ref[...]
    pb = blk.shape[1]
    nk = R // M
    nt = pb // TB

    iota = jax.lax.broadcasted_iota(jnp.int32, (M, TB), 0).astype(jnp.float32)
    rf = jnp.float32(R)
    cls_rows, plx_rows, ply_rows, plz_rows, bidx_rows = [], [], [], [], []
    for t in range(nt):
        tsl = slice(t * TB, (t + 1) * TB)
        bs = blk[0:1, tsl]
        xr = blk[1:2, tsl]
        yr = blk[2:3, tsl]
        zr = blk[3:4, tsl]
        fis = []
        for k in range(nk):
            sl = slice(k * M, (k + 1) * M)
            dx = xr - cx[sl]
            dy = yr - cy[sl]
            lx = dx * cc[sl] - dy * ss[sl]
            ly = dx * ss[sl] + dy * cc[sl]
            dz = zr - cz[sl]
            inb = ((jnp.abs(lx) <= hx[sl]) & (jnp.abs(ly) <= hy[sl])
                   & (jnp.abs(dz) <= hz[sl]))
            cand = jnp.where(inb, iota, rf)
            mn = jnp.min(cand, axis=0, keepdims=True)
            fis.append(jnp.where(mn < M, mn + (k * M), rf))
        fif = fis[nk - 1]
        for k in range(nk - 2, -1, -1):
            fif = jnp.where(bs == jnp.float32(k), fis[k], fif)
        fg = fif < rf
        fi = fif.astype(jnp.int32)

        G = jnp.zeros((16, TB), jnp.float32)
        for k in range(nk):
            ohf = (iota == (fif - jnp.float32(k * M))).astype(jnp.float32)
            G = G + jax.lax.dot_general(
                wg_ref[:, k * M:(k + 1) * M], ohf, (((1,), (0,)), ((), ())),
                precision=jax.lax.Precision.HIGHEST,
                preferred_element_type=jnp.float32)

        px = xr - G[0:1, :]
        py = yr - G[1:2, :]
        pz = zr - G[2:3, :]
        gc = G[3:4, :]
        gs = G[4:5, :]
        rx = px * gc - py * gs
        ry = px * gs + py * gc
        cls_rows.append(jnp.where(fg, G[8:9, :].astype(jnp.int32) + 1, 0))
        plx_rows.append(jnp.where(fg, rx / G[5:6, :] + 0.5, 0.0))
        ply_rows.append(jnp.where(fg, ry / G[6:7, :] + 0.5, 0.0))
        plz_rows.append(jnp.where(fg, pz / G[7:8, :] + 0.5, 0.0))
        bidx_rows.append(jnp.where(fg, fi, -1))

    cls = jnp.concatenate(cls_rows, axis=1)
    plx = jnp.concatenate(plx_rows, axis=1)
    ply = jnp.concatenate(ply_rows, axis=1)
    plz = jnp.concatenate(plz_rows, axis=1)
    bidx = jnp.concatenate(bidx_rows, axis=1)

    cls_ref[...] = cls.reshape(1, 1, cls.shape[-1])
    plx_ref[...] = plx.reshape(1, 1, plx.shape[-1])
    ply_ref[...] = ply.reshape(1, 1, ply.shape[-1])
    plz_ref[...] = plz.reshape(1, 1, plz.shape[-1])
    bidx_ref[...] = bidx.reshape(1, 1, bidx.shape[-1])


@functools.partial(jax.jit, static_argnames=())
def kernel(points, gt_boxes):
    n = points.shape[0]
    b, m, _ = gt_boxes.shape
    r = b * m
    pb = PB if n % PB == 0 else n
    g = n // pb

    ptsT = jnp.transpose(points)
    gb = gt_boxes.reshape(r, 8)
    gbT = jnp.transpose(gb)

    grid = (g,)
    out_shapes = [
        jax.ShapeDtypeStruct((g, 1, pb), jnp.int32),
        jax.ShapeDtypeStruct((g, 1, pb), jnp.float32),
        jax.ShapeDtypeStruct((g, 1, pb), jnp.float32),
        jax.ShapeDtypeStruct((g, 1, pb), jnp.float32),
        jax.ShapeDtypeStruct((g, 1, pb), jnp.int32),
    ]
    out_specs = [pl.BlockSpec((1, 1, pb), lambda i: (i, 0, 0))
                 for _ in range(5)]
    in_specs = [
        pl.BlockSpec((4, pb), lambda i: (0, i)),
        pl.BlockSpec((r, 8), lambda i: (0, 0)),
        pl.BlockSpec((8, r), lambda i: (0, 0)),
    ]
    scratch = [
        pltpu.VMEM((r, 16), jnp.float32),
        pltpu.VMEM((16, r), jnp.float32),
    ]
    cls, plx, ply, plz, bidx = pl.pallas_call(
        _body,
        grid=grid,
        in_specs=in_specs,
        out_specs=out_specs,
        out_shape=out_shapes,
        scratch_shapes=scratch,
    )(ptsT, gb, gbT)

    part = jnp.concatenate(
        [plx.reshape(n, 1), ply.reshape(n, 1), plz.reshape(n, 1)], axis=1)
    return cls.reshape(n), part, bidx.reshape(n)
